# baseline (speedup 1.0000x reference)
import functools

import jax
import jax.numpy as jnp
from jax import lax
from jax.experimental import pallas as pl
from jax.experimental.pallas import tpu as pltpu
from jax.experimental.pallas import tpu_sc as plsc

_N_USERS = 1000000
_EMB = 64
_BATCH = 16384

_NC = 2
_NS = 16
_L = 16
_NW = _NC * _NS
_BPW = _BATCH // _NW
_ICHUNK = 128
_NCHUNK = _BPW // _ICHUNK
_GROUPS = _BPW // _L


def _dot_kernel(table_hbm, idxu_hbm, idxi_hbm, out_hbm,
                idxu_v, idxi_v, rows_u, rows_i, out_v, sem_u, sem_i):
    wid = lax.axis_index("s") * _NC + lax.axis_index("c")
    base = wid * _BPW

    pltpu.sync_copy(idxu_hbm.at[pl.ds(wid * _NCHUNK, _NCHUNK)], idxu_v)
    pltpu.sync_copy(idxi_hbm.at[pl.ds(wid * _NCHUNK, _NCHUNK)], idxi_v)

    copies = []
    for j in range(_NCHUNK):
        copies.append(pltpu.async_copy(
            table_hbm.at[idxu_v.at[j]],
            rows_u.at[pl.ds(j * _ICHUNK, _ICHUNK)], sem_u))
        copies.append(pltpu.async_copy(
            table_hbm.at[idxi_v.at[j]],
            rows_i.at[pl.ds(j * _ICHUNK, _ICHUNK)], sem_i))
    for c in copies:
        c.wait()

    lanes = lax.iota(jnp.int32, _L)

    def group_body(g, carry):
        row = g * _L + lanes
        acc = jnp.zeros((_L,), jnp.float32)
        for k in range(_EMB):
            col = jnp.full((_L,), k, jnp.int32)
            u = plsc.load_gather(rows_u, [row, col])
            v = plsc.load_gather(rows_i, [row, col])
            acc = acc + u * v
        out_v[pl.ds(g * _L, _L)] = acc
        return carry

    lax.fori_loop(0, _GROUPS, group_body, 0)

    pltpu.sync_copy(out_v, out_hbm.at[pl.ds(base, _BPW)])


@jax.jit
def kernel(x_batch, table):
    idx = x_batch.astype(jnp.int32)
    idx_u = idx[:, 0].reshape(_NW * _NCHUNK, _ICHUNK)
    idx_i = (idx[:, 1] + _N_USERS).reshape(_NW * _NCHUNK, _ICHUNK)

    mesh = plsc.VectorSubcoreMesh(core_axis_name="c", subcore_axis_name="s")
    run = functools.partial(
        pl.kernel,
        mesh=mesh,
        compiler_params=pltpu.CompilerParams(
            needs_layout_passes=False, use_tc_tiling_on_sc=False),
        out_type=jax.ShapeDtypeStruct((_BATCH,), jnp.float32),
        scratch_types=[
            pltpu.VMEM((_NCHUNK, _ICHUNK), jnp.int32),
            pltpu.VMEM((_NCHUNK, _ICHUNK), jnp.int32),
            pltpu.VMEM((_BPW, _EMB), jnp.float32),
            pltpu.VMEM((_BPW, _EMB), jnp.float32),
            pltpu.VMEM((_BPW,), jnp.float32),
            pltpu.SemaphoreType.DMA,
            pltpu.SemaphoreType.DMA,
        ],
    )(_dot_kernel)
    out = run(table, idx_u, idx_i)
    return out.reshape(_BATCH, 1, 1)
